# Initial kernel scaffold; baseline (speedup 1.0000x reference)
import jax
import jax.numpy as jnp
from jax.experimental import pallas as pl


def kernel(x, edge_index, params):
    raise NotImplementedError("write your pallas kernel here")


# speedup vs baseline: 2.3858x; 2.3858x over previous
import functools

import numpy as np
import jax
import jax.numpy as jnp
from jax import lax
from jax.experimental import pallas as pl
from jax.experimental.pallas import tpu as pltpu
from jax.experimental.pallas import tpu_sc as plsc

NN = 10000
NNP = 10240
NE = 320000
DF = 64
NC = 4
K_IT = 10
GAMMA = 1.0 / (2.0 * (1.0 - 0.1))
CO = np.float32(GAMMA * 2.0 * (1.0 - 0.1))
LAM = np.float32(GAMMA * 0.1)

CHUNK = 128
NCH = NE // CHUNK
NWORK = 32
NPT = NNP // 16
EB = 4000
NB = 2000

_SC_MESH = dict(core_axis_name="c", subcore_axis_name="s")


def _elu(v):
    return jnp.where(v > 0, v, jnp.exp(jnp.minimum(v, 0.0)) - 1.0)


def _nchunks(wid):
    return 78 + jnp.where(wid < 4, 1, 0)


def _sc_gather(t1, t2, srcv, dstv):
    mesh = plsc.VectorSubcoreMesh(**_SC_MESH)

    @functools.partial(
        pl.kernel,
        out_type=(jax.ShapeDtypeStruct((NE, 128), jnp.float32),
                  jax.ShapeDtypeStruct((NE, 128), jnp.float32)),
        mesh=mesh,
        scratch_types=[
            pltpu.VMEM((CHUNK,), jnp.int32),
            pltpu.VMEM((CHUNK,), jnp.int32),
            pltpu.VMEM((CHUNK, 128), jnp.float32),
            pltpu.VMEM((CHUNK, 128), jnp.float32),
            pltpu.SemaphoreType.DMA,
            pltpu.SemaphoreType.DMA,
        ],
    )
    def k(ta, tb, sv, dv, ga, gb, siv, div, bufa, bufb, sema, semb):
        wid = lax.axis_index("c") * 16 + lax.axis_index("s")

        def body(j, carry):
            c = wid + NWORK * j
            pltpu.sync_copy(sv.at[c], siv)
            pltpu.sync_copy(dv.at[c], div)
            ca = pltpu.async_copy(ta.at[siv], bufa, sema)
            cb = pltpu.async_copy(tb.at[div], bufb, semb)
            ca.wait()
            cb.wait()
            pltpu.sync_copy(bufa, ga.at[pl.ds(c * CHUNK, CHUNK)])
            pltpu.sync_copy(bufb, gb.at[pl.ds(c * CHUNK, CHUNK)])
            return carry

        lax.fori_loop(0, _nchunks(wid), body, 0)

    return k(t1, t2, srcv, dstv)


def _sc_scatter(st, dstv, zeros):
    mesh = plsc.VectorSubcoreMesh(**_SC_MESH)

    @functools.partial(
        pl.kernel,
        out_type=jax.ShapeDtypeStruct((2, NNP, 128), jnp.float32),
        mesh=mesh,
        scratch_types=[
            pltpu.VMEM((CHUNK,), jnp.int32),
            pltpu.VMEM((CHUNK, 128), jnp.float32),
            pltpu.VMEM_SHARED((NNP, 128), jnp.float32),
        ],
    )
    def k(str_, dv, zz, out, div, buf, acc):
        cid = lax.axis_index("c")
        sid = lax.axis_index("s")
        wid = cid * 16 + sid
        pltpu.sync_copy(zz, acc.at[pl.ds(sid * NPT, NPT)])
        plsc.subcore_barrier()

        def body(j, carry):
            c = wid + NWORK * j
            pltpu.sync_copy(dv.at[c], div)
            pltpu.sync_copy(str_.at[pl.ds(c * CHUNK, CHUNK)], buf)
            pltpu.sync_copy(buf, acc.at[div], add=True)
            return carry

        lax.fori_loop(0, _nchunks(wid), body, 0)
        plsc.subcore_barrier()
        pltpu.sync_copy(acc.at[pl.ds(sid * NPT, NPT)],
                        out.at[cid, pl.ds(sid * NPT, NPT)])

    return k(st, dstv, zeros)


def _sc_deg(dstv, ones_in, zeros):
    mesh = plsc.VectorSubcoreMesh(**_SC_MESH)

    @functools.partial(
        pl.kernel,
        out_type=jax.ShapeDtypeStruct((2, NNP, 128), jnp.float32),
        mesh=mesh,
        scratch_types=[
            pltpu.VMEM((CHUNK,), jnp.int32),
            pltpu.VMEM((CHUNK, 128), jnp.float32),
            pltpu.VMEM_SHARED((NNP, 128), jnp.float32),
        ],
    )
    def k(dv, ones_r, zz, out, div, buf, acc):
        cid = lax.axis_index("c")
        sid = lax.axis_index("s")
        wid = cid * 16 + sid
        pltpu.sync_copy(ones_r, buf)
        pltpu.sync_copy(zz, acc.at[pl.ds(sid * NPT, NPT)])
        plsc.subcore_barrier()

        def body(j, carry):
            c = wid + NWORK * j
            pltpu.sync_copy(dv.at[c], div)
            pltpu.sync_copy(buf, acc.at[div], add=True)
            return carry

        lax.fori_loop(0, _nchunks(wid), body, 0)
        plsc.subcore_barrier()
        pltpu.sync_copy(acc.at[pl.ds(sid * NPT, NPT)],
                        out.at[cid, pl.ds(sid * NPT, NPT)])

    return k(dstv, ones_in, zeros)


def _full(shape):
    return pl.BlockSpec(shape, lambda *a: tuple(0 for _ in shape))


def _tc_node_init(x, iW1, ib1, iW2, ib2, jW1, jb1, jW2, jb2, pW1a, pW1b, pb1):
    def body(x_r, iW1_r, ib1_r, iW2_r, ib2_r, jW1_r, jb1_r, jW2_r, jb2_r,
             pa_r, pb_r, pb1_r, T1_r, T2_r):
        h = _elu(jnp.dot(x_r[...], iW1_r[...], preferred_element_type=jnp.float32) + ib1_r[...])
        h = _elu(jnp.dot(h, iW2_r[...], preferred_element_type=jnp.float32) + ib2_r[...])
        h = _elu(jnp.dot(h, jW1_r[...], preferred_element_type=jnp.float32) + jb1_r[...])
        h = _elu(jnp.dot(h, jW2_r[...], preferred_element_type=jnp.float32) + jb2_r[...])
        A = jnp.dot(h, pa_r[...], preferred_element_type=jnp.float32) + pb1_r[...]
        B = jnp.dot(h, pb_r[...], preferred_element_type=jnp.float32)
        T1_r[...] = jnp.concatenate([A, B], axis=1)
        T2_r[...] = jnp.concatenate([B, A], axis=1)

    grid = NN // NB
    return pl.pallas_call(
        body,
        grid=(grid,),
        in_specs=[pl.BlockSpec((NB, 128), lambda i: (i, 0)),
                  _full((128, DF)), _full((1, DF)), _full((DF, DF)), _full((1, DF)),
                  _full((DF, DF)), _full((1, DF)), _full((DF, DF)), _full((1, DF)),
                  _full((DF, DF)), _full((DF, DF)), _full((1, DF))],
        out_specs=[pl.BlockSpec((NB, 128), lambda i: (i, 0)),
                   pl.BlockSpec((NB, 128), lambda i: (i, 0))],
        out_shape=[jax.ShapeDtypeStruct((NN, 128), jnp.float32),
                   jax.ShapeDtypeStruct((NN, 128), jnp.float32)],
    )(x, iW1, ib1, iW2, ib2, jW1, jb1, jW2, jb2, pW1a, pW1b, pb1)


def _tc_pre_edge(GA, GB, pW2, pb2):
    def body(ga_r, gb_r, w_r, b_r, hh_r, st_r):
        h1 = _elu(ga_r[...][:, :DF] + gb_r[...][:, :DF])
        hh = _elu(jnp.dot(h1, w_r[...], preferred_element_type=jnp.float32) + b_r[...])
        hh_r[...] = hh
        st_r[...] = jnp.concatenate([hh, jnp.zeros_like(hh)], axis=1)

    grid = NE // EB
    gspec = pl.BlockSpec((EB, 128), lambda i: (i, 0))
    return pl.pallas_call(
        body,
        grid=(grid,),
        in_specs=[gspec, gspec, _full((DF, DF)), _full((1, DF))],
        out_specs=[pl.BlockSpec((EB, DF), lambda i: (i, 0)), gspec],
        out_shape=[jax.ShapeDtypeStruct((NE, DF), jnp.float32),
                   jax.ShapeDtypeStruct((NE, 128), jnp.float32)],
    )(GA, GB, pW2, pb2)


def _tc_node_iter(parts, rdeg, nW1, nb1, nW2, nb2, xW1, xb1, xW2, xb2,
                  eW1a, eW1b, eb1):
    def body(p_r, rd_r, nW1_r, nb1_r, nW2_r, nb2_r, xW1_r, xb1_r, xW2_r,
             xb2_r, ea_r, eb_r, eb1_r, T1_r, T2_r):
        agg = (p_r[0, :, :DF] + p_r[1, :, :DF]) * rd_r[...]
        h = _elu(jnp.dot(agg, nW1_r[...], preferred_element_type=jnp.float32) + nb1_r[...])
        h = _elu(jnp.dot(h, nW2_r[...], preferred_element_type=jnp.float32) + nb2_r[...])
        h = _elu(jnp.dot(h, xW1_r[...], preferred_element_type=jnp.float32) + xb1_r[...])
        h = _elu(jnp.dot(h, xW2_r[...], preferred_element_type=jnp.float32) + xb2_r[...])
        P = jnp.dot(h, ea_r[...], preferred_element_type=jnp.float32) + eb1_r[...]
        Q = jnp.dot(h, eb_r[...], preferred_element_type=jnp.float32)
        T1_r[...] = jnp.concatenate([P, Q], axis=1)
        T2_r[...] = jnp.concatenate([Q, P], axis=1)

    grid = NN // NB
    return pl.pallas_call(
        body,
        grid=(grid,),
        in_specs=[pl.BlockSpec((2, NB, 128), lambda i: (0, i, 0)),
                  pl.BlockSpec((NB, 1), lambda i: (i, 0)),
                  _full((DF, DF)), _full((1, DF)), _full((DF, DF)), _full((1, DF)),
                  _full((DF, DF)), _full((1, DF)), _full((DF, DF)), _full((1, DF)),
                  _full((DF, DF)), _full((DF, DF)), _full((1, DF))],
        out_specs=[pl.BlockSpec((NB, 128), lambda i: (i, 0)),
                   pl.BlockSpec((NB, 128), lambda i: (i, 0))],
        out_shape=[jax.ShapeDtypeStruct((NN, 128), jnp.float32),
                   jax.ShapeDtypeStruct((NN, 128), jnp.float32)],
    )(parts, rdeg, nW1, nb1, nW2, nb2, xW1, xb1, xW2, xb2, eW1a, eW1b, eb1)


def _tc_edge_iter(GA, GB, st, eW1c, eW2, eb2):
    def body(ga_r, gb_r, st_r, w1_r, w2_r, b2_r, nst_r):
        stv = st_r[...]
        edv = stv[:, :DF]
        uv = stv[:, DF:]
        t = (ga_r[...][:, :DF] + gb_r[...][:, :DF]
             + jnp.dot(edv, w1_r[...], preferred_element_type=jnp.float32))
        h1 = _elu(t)
        ned = _elu(jnp.dot(h1, w2_r[...], preferred_element_type=jnp.float32) + b2_r[...])
        lx = edv - ned
        z = uv - CO * lx
        rn = jnp.sqrt(jnp.sum(z * z, axis=1, keepdims=True) + 1e-12)
        sc = jnp.maximum(rn - LAM, 0.0)
        sc = jnp.where(rn > 0, sc / rn, sc)
        nst_r[...] = jnp.concatenate([ned, sc * z], axis=1)

    grid = NE // EB
    gspec = pl.BlockSpec((EB, 128), lambda i: (i, 0))
    return pl.pallas_call(
        body,
        grid=(grid,),
        in_specs=[gspec, gspec, gspec,
                  _full((DF, DF)), _full((DF, DF)), _full((1, DF))],
        out_specs=gspec,
        out_shape=jax.ShapeDtypeStruct((NE, 128), jnp.float32),
    )(GA, GB, st, eW1c, eW2, eb2)


def _tc_head_stats(hh, st, W1s, b1s, W2s, b2s):
    def body(hh_r, st_r, W1_r, b1_r, W2_r, b2_r, s_r, q_r):
        @pl.when(pl.program_id(0) == 0)
        def _init():
            s_r[...] = jnp.zeros_like(s_r)
            q_r[...] = jnp.zeros_like(q_r)

        xv = hh_r[...] + st_r[...][:, DF:]
        ss, qq = [], []
        for c in range(NC):
            h1 = _elu(jnp.dot(xv, W1_r[c], preferred_element_type=jnp.float32) + b1_r[c])
            h2 = _elu(jnp.dot(h1, W2_r[c], preferred_element_type=jnp.float32) + b2_r[c])
            ss.append(jnp.sum(h2, axis=0, keepdims=True))
            qq.append(jnp.sum(h2 * h2, axis=0, keepdims=True))
        s_r[...] += jnp.concatenate(ss, axis=0)
        q_r[...] += jnp.concatenate(qq, axis=0)

    grid = NE // EB
    espec = pl.BlockSpec((EB, DF), lambda i: (i, 0))
    gspec = pl.BlockSpec((EB, 128), lambda i: (i, 0))
    return pl.pallas_call(
        body,
        grid=(grid,),
        in_specs=[espec, gspec,
                  _full((NC, DF, DF)), _full((NC, 1, DF)),
                  _full((NC, DF, DF)), _full((NC, 1, DF))],
        out_specs=[_full((NC, DF)), _full((NC, DF))],
        out_shape=[jax.ShapeDtypeStruct((NC, DF), jnp.float32),
                   jax.ShapeDtypeStruct((NC, DF), jnp.float32)],
        compiler_params=pltpu.CompilerParams(
            dimension_semantics=("arbitrary",)),
    )(hh, st, W1s, b1s, W2s, b2s)


def _tc_head_out(hh, st, W1s, b1s, W2s, b2s, Wf, bf):
    def body(hh_r, st_r, W1_r, b1_r, W2_r, b2_r, Wf_r, bf_r, o_r):
        xv = hh_r[...] + st_r[...][:, DF:]
        ys = []
        for c in range(NC):
            h1 = _elu(jnp.dot(xv, W1_r[c], preferred_element_type=jnp.float32) + b1_r[c])
            h2 = _elu(jnp.dot(h1, W2_r[c], preferred_element_type=jnp.float32) + b2_r[c])
            ys.append(jnp.dot(h2, Wf_r[c], preferred_element_type=jnp.float32))
        o_r[...] = jnp.concatenate(ys, axis=1) + bf_r[...]

    grid = NE // EB
    espec = pl.BlockSpec((EB, DF), lambda i: (i, 0))
    gspec = pl.BlockSpec((EB, 128), lambda i: (i, 0))
    return pl.pallas_call(
        body,
        grid=(grid,),
        in_specs=[espec, gspec,
                  _full((NC, DF, DF)), _full((NC, 1, DF)),
                  _full((NC, DF, DF)), _full((NC, 1, DF)),
                  _full((NC, DF, NC)), _full((1, NC * NC))],
        out_specs=pl.BlockSpec((EB, NC * NC), lambda i: (i, 0)),
        out_shape=jax.ShapeDtypeStruct((NE, NC * NC), jnp.float32),
    )(hh, st, W1s, b1s, W2s, b2s, Wf, bf)


def kernel(x, edge_index, params):
    src = edge_index[0].astype(jnp.int32)
    dst = edge_index[1].astype(jnp.int32)
    srcv = src.reshape(NCH, CHUNK)
    dstv = dst.reshape(NCH, CHUNK)

    p = params
    im0, im1 = p["initial_mlp"]
    pre = p["pre_blocks_mlp"]
    blk = p["blocks"][0]
    nmlp, xmlp, emlp = blk["node_mlp"], blk["extra"][0], blk["edge_mlp"]

    r1 = lambda b: b.reshape(1, -1)
    pW1a = pre["W1"][:DF]
    pW1b = pre["W1"][DF:]
    eW1a = emlp["W1"][:DF]
    eW1b = emlp["W1"][DF:2 * DF]
    eW1c = emlp["W1"][2 * DF:]

    T1, T2 = _tc_node_init(x, im0["W1"], r1(im0["b1"]), im0["W2"], r1(im0["b2"]),
                           im1["W1"], r1(im1["b1"]), im1["W2"], r1(im1["b2"]),
                           pW1a, pW1b, r1(pre["b1"]))

    zeros128 = jnp.zeros((NPT, 128), jnp.float32)
    ones128 = jnp.ones((CHUNK, 128), jnp.float32)
    degp = _sc_deg(dstv, ones128, zeros128)
    deg = degp[0, :NN, 0] + degp[1, :NN, 0]
    rdeg = (1.0 / jnp.clip(deg, 1.0, None)).reshape(NN, 1)

    GA0, GB0 = _sc_gather(T1, T2, srcv, dstv)
    hh, st = _tc_pre_edge(GA0, GB0, pre["W2"], r1(pre["b2"]))

    for it in range(K_IT):
        parts = _sc_scatter(st, dstv, zeros128)
        T1, T2 = _tc_node_iter(parts, rdeg,
                               nmlp["W1"], r1(nmlp["b1"]), nmlp["W2"], r1(nmlp["b2"]),
                               xmlp["W1"], r1(xmlp["b1"]), xmlp["W2"], r1(xmlp["b2"]),
                               eW1a, eW1b, r1(emlp["b1"]))
        GA, GB = _sc_gather(T1, T2, srcv, dstv)
        st = _tc_edge_iter(GA, GB, st, eW1c, emlp["W2"], r1(emlp["b2"]))

    W1s = jnp.stack([p["fact"][c]["W1"] for c in range(NC)])
    b1s = jnp.stack([p["fact"][c]["b1"].reshape(1, DF) for c in range(NC)])
    W2s = jnp.stack([p["fact"][c]["W2"] for c in range(NC)])
    b2s = jnp.stack([p["fact"][c]["b2"].reshape(1, DF) for c in range(NC)])
    sums, sqs = _tc_head_stats(hh, st, W1s, b1s, W2s, b2s)
    mu = sums / NE
    var = sqs / NE - mu * mu
    rstd = 1.0 / jnp.sqrt(var + 1e-5)
    scale = jnp.stack([p["fact"][c]["g"] for c in range(NC)]) * rstd
    beta = jnp.stack([p["fact"][c]["beta"] for c in range(NC)])
    shift = beta - mu * scale
    fcW = jnp.stack(p["fc_W"])
    Wf = scale[:, :, None] * fcW
    bf = (jnp.einsum("cd,cdo->co", shift, fcW) + jnp.stack(p["fc_b"])).reshape(1, NC * NC)
    return _tc_head_out(hh, st, W1s, b1s, W2s, b2s, Wf, bf)
